# baseline (speedup 1.0000x reference)
import functools

import jax
import jax.numpy as jnp
from jax import lax
from jax.experimental import pallas as pl
from jax.experimental.pallas import tpu as pltpu
from jax.experimental.pallas import tpu_sc as plsc

NC = 2
NS = 16
NW = NC * NS
K = 80
CW = 16
NEG_INF = float("-inf")
G = 64
BM = 1000


def _sc_aggregate_fn(N, D, E):
  e_per_tile = (E + NW - 1) // NW
  KC = 128
  cph = 40
  cpt = ((e_per_tile + KC - 1) // KC + cph - 1) // cph * cph
  phases = cpt // cph
  C = NW * cpt
  pairs_ph = cph // 2
  assert cph % 2 == 0 and pairs_ph >= 3
  Np = ((N + 1 + 8 * NS - 1) // (8 * NS)) * (8 * NS)
  rows_per_tile = Np // NS
  zrows = 8
  assert rows_per_tile % zrows == 0
  mesh = plsc.VectorSubcoreMesh(core_axis_name="c", subcore_axis_name="s")

  out_type = jax.ShapeDtypeStruct((NC * Np, D), jnp.float32)
  scratch = [
      pltpu.VMEM((cph, KC), jnp.int32),
      pltpu.VMEM((cph, KC), jnp.int32),
      pltpu.VMEM((KC, D), jnp.float32),
      pltpu.VMEM((KC, D), jnp.float32),
      pltpu.VMEM((zrows, D), jnp.float32),
      pltpu.VMEM_SHARED((Np, D), jnp.float32),
      pltpu.SemaphoreType.DMA,
      pltpu.SemaphoreType.DMA,
      pltpu.SemaphoreType.DMA,
      pltpu.SemaphoreType.DMA,
  ]

  def body(table, src2, dst2, out, idx_s2, idx_d2,
           rows0, rows1, zbuf, acc, gsem0, gsem1, ssem0, ssem1):
    rows = (rows0, rows1)
    gsem = (gsem0, gsem1)
    ssem = (ssem0, ssem1)
    c = lax.axis_index("c")
    s = lax.axis_index("s")
    wid = s * NC + c
    zero16 = jnp.zeros((16,), jnp.float32)

    @pl.loop(0, zrows)
    def _(r):
      for col in range(D // 16):
        zbuf[r, pl.ds(col * 16, 16)] = zero16

    row0 = s * rows_per_tile

    @pl.loop(0, rows_per_tile // zrows)
    def _(i):
      pltpu.sync_copy(zbuf, acc.at[pl.ds(row0 + i * zrows, zrows), :])
    plsc.subcore_barrier()

    cbase = wid * cpt

    for ph in range(phases):
      pltpu.sync_copy(src2.at[pl.ds(cbase + ph * cph, cph), :], idx_s2)
      pltpu.sync_copy(dst2.at[pl.ds(cbase + ph * cph, cph), :], idx_d2)

      @pl.loop(0, pairs_ph)
      def _(m):
        g0 = pltpu.async_copy(table.at[idx_s2.at[2 * m]], rows0, gsem0)
        g1 = pltpu.async_copy(table.at[idx_s2.at[2 * m + 1]], rows1, gsem1)
        g0.wait()
        g1.wait()
        s0 = pltpu.async_copy(rows0, acc.at[idx_d2.at[2 * m]], ssem0,
                              add=True)
        s1 = pltpu.async_copy(rows1, acc.at[idx_d2.at[2 * m + 1]], ssem1,
                              add=True)
        s0.wait()
        s1.wait()

        plsc.subcore_barrier()

    pltpu.sync_copy(acc.at[pl.ds(row0, rows_per_tile), :],
                    out.at[pl.ds(c * Np + row0, rows_per_tile), :])

  return pl.kernel(body, out_type=out_type, mesh=mesh,
                   scratch_types=scratch), Np, C * KC, KC


def _mm2_body(x_ref, wa_ref, wb_ref, oa_ref, ob_ref):
  xv = x_ref[...]
  oa_ref[...] = jnp.dot(xv, wa_ref[...], preferred_element_type=jnp.float32)
  ob_ref[...] = jnp.dot(xv, wb_ref[...], preferred_element_type=jnp.float32)


def _mm2(x, wa, wb):
  n, d = x.shape
  da, db = wa.shape[1], wb.shape[1]
  return pl.pallas_call(
      _mm2_body,
      grid=(n // BM,),
      in_specs=[
          pl.BlockSpec((BM, d), lambda i: (i, 0)),
          pl.BlockSpec((d, da), lambda i: (0, 0)),
          pl.BlockSpec((d, db), lambda i: (0, 0)),
      ],
      out_specs=[
          pl.BlockSpec((BM, da), lambda i: (i, 0)),
          pl.BlockSpec((BM, db), lambda i: (i, 0)),
      ],
      out_shape=[
          jax.ShapeDtypeStruct((n, da), jnp.float32),
          jax.ShapeDtypeStruct((n, db), jnp.float32),
      ],
  )(x, wa, wb)


def _combine_body(agg_ref, cnt_ref, r_ref, wa_ref, wb_ref,
                  x1_ref, ta_ref, tb_ref):
  ssum = agg_ref[0] + agg_ref[1]
  cnt = jnp.maximum(cnt_ref[0, :, 0:1] + cnt_ref[1, :, 0:1], 1.0)
  x1 = jnp.maximum(ssum / cnt + r_ref[...], 0.0)
  x1_ref[...] = x1
  ta_ref[...] = jnp.dot(x1, wa_ref[...], preferred_element_type=jnp.float32)
  tb_ref[...] = jnp.dot(x1, wb_ref[...], preferred_element_type=jnp.float32)


def _combine(agg, cnt, r, wa, wb):
  n, d = r.shape
  return pl.pallas_call(
      _combine_body,
      grid=(n // BM,),
      in_specs=[
          pl.BlockSpec((2, BM, d), lambda i: (0, i, 0)),
          pl.BlockSpec((2, BM, d), lambda i: (0, i, 0)),
          pl.BlockSpec((BM, d), lambda i: (i, 0)),
          pl.BlockSpec((d, d), lambda i: (0, 0)),
          pl.BlockSpec((d, d), lambda i: (0, 0)),
      ],
      out_specs=[
          pl.BlockSpec((BM, d), lambda i: (i, 0)),
          pl.BlockSpec((BM, d), lambda i: (i, 0)),
          pl.BlockSpec((BM, d), lambda i: (i, 0)),
      ],
      out_shape=[
          jax.ShapeDtypeStruct((n, d), jnp.float32),
          jax.ShapeDtypeStruct((n, d), jnp.float32),
          jax.ShapeDtypeStruct((n, d), jnp.float32),
      ],
  )(agg, cnt, r, wa, wb)


def _final_body(agg_ref, cnt_ref, r_ref, x1_ref, b_ref, o_ref):
  @pl.when(pl.program_id(0) == 0)
  def _():
    o_ref[...] = jnp.full(o_ref.shape, NEG_INF, jnp.float32)

  ssum = agg_ref[0] + agg_ref[1]
  cnt = jnp.maximum(cnt_ref[0, :, 0:1] + cnt_ref[1, :, 0:1], 1.0)
  x2 = jnp.maximum(ssum / cnt + r_ref[...], 0.0)
  xs = jnp.concatenate([x1_ref[...], x2], axis=1)
  b = b_ref[...]
  gmin = b_ref[0, 0]
  gmax = b_ref[b_ref.shape[0] - 1, 0]
  for g in range(G):
    @pl.when(jnp.logical_and(gmin <= g, g <= gmax))
    def _(g=g):
      col = jnp.max(jnp.where(b == g, xs, NEG_INF), axis=0)
      o_ref[g, :] = jnp.maximum(o_ref[g, :], col)


def _final(agg, cnt, r, x1, batch2):
  n, d = r.shape
  return pl.pallas_call(
      _final_body,
      grid=(n // BM,),
      in_specs=[
          pl.BlockSpec((2, BM, d), lambda i: (0, i, 0)),
          pl.BlockSpec((2, BM, d), lambda i: (0, i, 0)),
          pl.BlockSpec((BM, d), lambda i: (i, 0)),
          pl.BlockSpec((BM, d), lambda i: (i, 0)),
          pl.BlockSpec((BM, 1), lambda i: (i, 0)),
      ],
      out_specs=pl.BlockSpec((G, 2 * d), lambda i: (0, 0)),
      out_shape=jax.ShapeDtypeStruct((G, 2 * d), jnp.float32),
  )(agg, cnt, r, x1, batch2)


def kernel(x, edge_index, batch, W1_l, W1_r, W2_l, W2_r):
  n, d = x.shape
  e = edge_index.shape[1]

  agg, np_pad, e_pad, kc = _sc_aggregate_fn(n, d, e)
  src = jnp.concatenate(
      [edge_index[0], jnp.zeros((e_pad - e,), jnp.int32)]).reshape(-1, kc)
  dst = jnp.concatenate(
      [edge_index[1], jnp.full((e_pad - e,), n, jnp.int32)]).reshape(-1, kc)

  def as3d(a):
    return a.reshape(NC, np_pad, d)

  t1, r1 = _mm2(x, W1_l.T, W1_r.T)
  agg1 = as3d(agg(t1, src, dst))
  cnt = as3d(agg(jnp.ones((n, d), jnp.float32), src, dst))
  x1, t2, r2 = _combine(agg1, cnt, r1, W2_l.T, W2_r.T)
  agg2 = as3d(agg(t2, src, dst))
  return _final(agg2, cnt, r2, x1, batch.reshape(n, 1))


# speedup vs baseline: 1.7962x; 1.7962x over previous
import jax
import jax.numpy as jnp
from jax import lax
from jax.experimental import pallas as pl
from jax.experimental.pallas import tpu as pltpu
from jax.experimental.pallas import tpu_sc as plsc

NC = 2
NS = 16
NW = NC * NS
K = 80
CW = 16
NEG_INF = float("-inf")
G = 64
BM = 1000


def _sc_aggregate_fn(N, D, E):
  e_per_tile = E // NW
  chunks = e_per_tile // K
  assert e_per_tile * NW == E and chunks * K == e_per_tile
  Np = ((N + 8 * NS - 1) // (8 * NS)) * (8 * NS)
  rows_per_tile = Np // NS
  zrows = 8
  assert rows_per_tile % zrows == 0
  mesh = plsc.VectorSubcoreMesh(core_axis_name="c", subcore_axis_name="s")

  out_type = jax.ShapeDtypeStruct((NC * Np, D), jnp.float32)
  scratch = [
      pltpu.VMEM((K,), jnp.int32),
      pltpu.VMEM((K,), jnp.int32),
      pltpu.VMEM((K, D), jnp.float32),
      pltpu.VMEM((zrows, D), jnp.float32),
      pltpu.VMEM_SHARED((Np, D), jnp.float32),
      pltpu.SemaphoreType.DMA,
  ]

  def body(table, src, dst, out, idx_s, idx_d, rows, zbuf, acc, sem):
    c = lax.axis_index("c")
    s = lax.axis_index("s")
    wid = s * NC + c
    zero16 = jnp.zeros((16,), jnp.float32)

    @pl.loop(0, zrows)
    def _(r):
      for col in range(D // 16):
        zbuf[r, pl.ds(col * 16, 16)] = zero16

    row0 = s * rows_per_tile

    @pl.loop(0, rows_per_tile // zrows)
    def _(i):
      pltpu.sync_copy(zbuf, acc.at[pl.ds(row0 + i * zrows, zrows), :])
    plsc.subcore_barrier()

    ebase = wid * e_per_tile

    @pl.loop(0, chunks)
    def _(j):
      base = ebase + j * K
      pltpu.sync_copy(src.at[pl.ds(base, K)], idx_s)
      pltpu.sync_copy(dst.at[pl.ds(base, K)], idx_d)
      pltpu.async_copy(table.at[idx_s], rows, sem).wait()
      pltpu.sync_copy(rows, acc.at[idx_d], add=True)
    plsc.subcore_barrier()

    pltpu.sync_copy(acc.at[pl.ds(row0, rows_per_tile), :],
                    out.at[pl.ds(c * Np + row0, rows_per_tile), :])

  return pl.kernel(body, out_type=out_type, mesh=mesh,
                   scratch_types=scratch), Np


def _mm2_body(x_ref, wa_ref, wb_ref, oa_ref, ob_ref):
  xv = x_ref[...]
  oa_ref[...] = jnp.dot(xv, wa_ref[...], preferred_element_type=jnp.float32)
  ob_ref[...] = jnp.dot(xv, wb_ref[...], preferred_element_type=jnp.float32)


def _mm2(x, wa, wb):
  n, d = x.shape
  da, db = wa.shape[1], wb.shape[1]
  return pl.pallas_call(
      _mm2_body,
      grid=(n // BM,),
      in_specs=[
          pl.BlockSpec((BM, d), lambda i: (i, 0)),
          pl.BlockSpec((d, da), lambda i: (0, 0)),
          pl.BlockSpec((d, db), lambda i: (0, 0)),
      ],
      out_specs=[
          pl.BlockSpec((BM, da), lambda i: (i, 0)),
          pl.BlockSpec((BM, db), lambda i: (i, 0)),
      ],
      out_shape=[
          jax.ShapeDtypeStruct((n, da), jnp.float32),
          jax.ShapeDtypeStruct((n, db), jnp.float32),
      ],
  )(x, wa, wb)


def _combine_body(agg_ref, cnt_ref, r_ref, wa_ref, wb_ref,
                  x1_ref, ta_ref, tb_ref):
  ssum = agg_ref[0] + agg_ref[1]
  cnt = jnp.maximum(cnt_ref[0, :, 0:1] + cnt_ref[1, :, 0:1], 1.0)
  x1 = jnp.maximum(ssum / cnt + r_ref[...], 0.0)
  x1_ref[...] = x1
  ta_ref[...] = jnp.dot(x1, wa_ref[...], preferred_element_type=jnp.float32)
  tb_ref[...] = jnp.dot(x1, wb_ref[...], preferred_element_type=jnp.float32)


def _combine(agg, cnt, r, wa, wb):
  n, d = r.shape
  return pl.pallas_call(
      _combine_body,
      grid=(n // BM,),
      in_specs=[
          pl.BlockSpec((2, BM, d), lambda i: (0, i, 0)),
          pl.BlockSpec((2, BM, d), lambda i: (0, i, 0)),
          pl.BlockSpec((BM, d), lambda i: (i, 0)),
          pl.BlockSpec((d, d), lambda i: (0, 0)),
          pl.BlockSpec((d, d), lambda i: (0, 0)),
      ],
      out_specs=[
          pl.BlockSpec((BM, d), lambda i: (i, 0)),
          pl.BlockSpec((BM, d), lambda i: (i, 0)),
          pl.BlockSpec((BM, d), lambda i: (i, 0)),
      ],
      out_shape=[
          jax.ShapeDtypeStruct((n, d), jnp.float32),
          jax.ShapeDtypeStruct((n, d), jnp.float32),
          jax.ShapeDtypeStruct((n, d), jnp.float32),
      ],
  )(agg, cnt, r, wa, wb)


def _final_body(agg_ref, cnt_ref, r_ref, x1_ref, b_ref, o_ref):
  @pl.when(pl.program_id(0) == 0)
  def _():
    o_ref[...] = jnp.full(o_ref.shape, NEG_INF, jnp.float32)

  ssum = agg_ref[0] + agg_ref[1]
  cnt = jnp.maximum(cnt_ref[0, :, 0:1] + cnt_ref[1, :, 0:1], 1.0)
  x2 = jnp.maximum(ssum / cnt + r_ref[...], 0.0)
  xs = jnp.concatenate([x1_ref[...], x2], axis=1)
  b = b_ref[...]
  gmin = b_ref[0, 0]
  gmax = b_ref[b_ref.shape[0] - 1, 0]
  for g in range(G):
    @pl.when(jnp.logical_and(gmin <= g, g <= gmax))
    def _(g=g):
      col = jnp.max(jnp.where(b == g, xs, NEG_INF), axis=0)
      o_ref[g, :] = jnp.maximum(o_ref[g, :], col)


def _final(agg, cnt, r, x1, batch2):
  n, d = r.shape
  return pl.pallas_call(
      _final_body,
      grid=(n // BM,),
      in_specs=[
          pl.BlockSpec((2, BM, d), lambda i: (0, i, 0)),
          pl.BlockSpec((2, BM, d), lambda i: (0, i, 0)),
          pl.BlockSpec((BM, d), lambda i: (i, 0)),
          pl.BlockSpec((BM, d), lambda i: (i, 0)),
          pl.BlockSpec((BM, 1), lambda i: (i, 0)),
      ],
      out_specs=pl.BlockSpec((G, 2 * d), lambda i: (0, 0)),
      out_shape=jax.ShapeDtypeStruct((G, 2 * d), jnp.float32),
  )(agg, cnt, r, x1, batch2)


def kernel(x, edge_index, batch, W1_l, W1_r, W2_l, W2_r):
  n, d = x.shape
  e = edge_index.shape[1]
  src = edge_index[0]
  dst = edge_index[1]

  agg, np_pad = _sc_aggregate_fn(n, d, e)

  def as3d(a):
    return a.reshape(NC, np_pad, d)

  t1, r1 = _mm2(x, W1_l.T, W1_r.T)
  agg1 = as3d(agg(t1, src, dst))
  cnt = as3d(agg(jnp.ones((n, d), jnp.float32), src, dst))
  x1, t2, r2 = _combine(agg1, cnt, r1, W2_l.T, W2_r.T)
  agg2 = as3d(agg(t2, src, dst))
  return _final(agg2, cnt, r2, x1, batch.reshape(n, 1))


# speedup vs baseline: 2.2009x; 1.2253x over previous
import jax
import jax.numpy as jnp
from jax import lax
from jax.experimental import pallas as pl
from jax.experimental.pallas import tpu as pltpu
from jax.experimental.pallas import tpu_sc as plsc

NC = 2
NS = 16
NW = NC * NS
K = 80
CW = 16
NEG_INF = float("-inf")
G = 64
BM = 1000


def _sc_aggregate_fn(N, D, E):
  e_per_tile = E // NW
  chunks = e_per_tile // K
  assert e_per_tile * NW == E and chunks * K == e_per_tile
  Np = ((N + 8 * NS - 1) // (8 * NS)) * (8 * NS)
  rows_per_tile = Np // NS
  zrows = 8
  assert rows_per_tile % zrows == 0
  mesh = plsc.VectorSubcoreMesh(core_axis_name="c", subcore_axis_name="s")

  out_type = jax.ShapeDtypeStruct((NC * Np, D), jnp.float32)
  scratch = [
      pltpu.VMEM((K,), jnp.int32),
      pltpu.VMEM((K,), jnp.int32),
      pltpu.VMEM((K, D), jnp.float32),
      pltpu.VMEM((zrows, D), jnp.float32),
      pltpu.VMEM_SHARED((Np, D), jnp.float32),
      pltpu.SemaphoreType.DMA,
  ]

  def body(table, src, dst, out, idx_s, idx_d, rows, zbuf, acc, sem):
    c = lax.axis_index("c")
    s = lax.axis_index("s")
    wid = s * NC + c
    zero16 = jnp.zeros((16,), jnp.float32)

    @pl.loop(0, zrows)
    def _(r):
      for col in range(D // 16):
        zbuf[r, pl.ds(col * 16, 16)] = zero16

    row0 = s * rows_per_tile

    @pl.loop(0, rows_per_tile // zrows)
    def _(i):
      pltpu.sync_copy(zbuf, acc.at[pl.ds(row0 + i * zrows, zrows), :])
    plsc.subcore_barrier()

    ebase = wid * e_per_tile

    @pl.loop(0, chunks)
    def _(j):
      base = ebase + j * K
      pltpu.sync_copy(src.at[pl.ds(base, K)], idx_s)
      pltpu.sync_copy(dst.at[pl.ds(base, K)], idx_d)
      pltpu.async_copy(table.at[idx_s], rows, sem).wait()
      pltpu.sync_copy(rows, acc.at[idx_d], add=True)
    plsc.subcore_barrier()

    pltpu.sync_copy(acc.at[pl.ds(row0, rows_per_tile), :],
                    out.at[pl.ds(c * Np + row0, rows_per_tile), :])

  return pl.kernel(body, out_type=out_type, mesh=mesh,
                   scratch_types=scratch), Np


def _sc_count_fn(N, E):
  D = 128
  e_per_tile = E // NW
  chunks = e_per_tile // K
  assert e_per_tile * NW == E and chunks * K == e_per_tile
  Np = ((N + 8 * NS - 1) // (8 * NS)) * (8 * NS)
  rows_per_tile = Np // NS
  zrows = 8
  assert rows_per_tile % zrows == 0
  mesh = plsc.VectorSubcoreMesh(core_axis_name="c", subcore_axis_name="s")

  out_type = jax.ShapeDtypeStruct((NC * Np, D), jnp.float32)
  scratch = [
      pltpu.VMEM((K,), jnp.int32),
      pltpu.VMEM((K, D), jnp.float32),
      pltpu.VMEM((zrows, D), jnp.float32),
      pltpu.VMEM_SHARED((Np, D), jnp.float32),
  ]

  def body(src, dst, out, idx_d, ones, zbuf, acc):
    c = lax.axis_index("c")
    s = lax.axis_index("s")
    wid = s * NC + c
    zero16 = jnp.zeros((16,), jnp.float32)
    one16 = jnp.ones((16,), jnp.float32)

    @pl.loop(0, zrows)
    def _(r):
      for col in range(D // 16):
        zbuf[r, pl.ds(col * 16, 16)] = zero16

    @pl.loop(0, K)
    def _(r):
      for col in range(D // 16):
        ones[r, pl.ds(col * 16, 16)] = one16

    row0 = s * rows_per_tile

    @pl.loop(0, rows_per_tile // zrows)
    def _(i):
      pltpu.sync_copy(zbuf, acc.at[pl.ds(row0 + i * zrows, zrows), :])
    plsc.subcore_barrier()

    ebase = wid * e_per_tile

    @pl.loop(0, chunks)
    def _(j):
      pltpu.sync_copy(dst.at[pl.ds(ebase + j * K, K)], idx_d)
      pltpu.sync_copy(ones, acc.at[idx_d], add=True)
    plsc.subcore_barrier()

    pltpu.sync_copy(acc.at[pl.ds(row0, rows_per_tile), :],
                    out.at[pl.ds(c * Np + row0, rows_per_tile), :])

  return pl.kernel(body, out_type=out_type, mesh=mesh,
                   scratch_types=scratch)


def _mm2_body(x_ref, wa_ref, wb_ref, oa_ref, ob_ref):
  xv = x_ref[...]
  oa_ref[...] = jnp.dot(xv, wa_ref[...], preferred_element_type=jnp.float32)
  ob_ref[...] = jnp.dot(xv, wb_ref[...], preferred_element_type=jnp.float32)


def _mm2(x, wa, wb):
  n, d = x.shape
  da, db = wa.shape[1], wb.shape[1]
  return pl.pallas_call(
      _mm2_body,
      grid=(n // BM,),
      in_specs=[
          pl.BlockSpec((BM, d), lambda i: (i, 0)),
          pl.BlockSpec((d, da), lambda i: (0, 0)),
          pl.BlockSpec((d, db), lambda i: (0, 0)),
      ],
      out_specs=[
          pl.BlockSpec((BM, da), lambda i: (i, 0)),
          pl.BlockSpec((BM, db), lambda i: (i, 0)),
      ],
      out_shape=[
          jax.ShapeDtypeStruct((n, da), jnp.float32),
          jax.ShapeDtypeStruct((n, db), jnp.float32),
      ],
  )(x, wa, wb)


def _combine_body(agg_ref, cnt_ref, r_ref, wa_ref, wb_ref,
                  x1_ref, ta_ref, tb_ref):
  ssum = agg_ref[0] + agg_ref[1]
  cnt = jnp.maximum(cnt_ref[0, :, 0:1] + cnt_ref[1, :, 0:1], 1.0)
  x1 = jnp.maximum(ssum / cnt + r_ref[...], 0.0)
  x1_ref[...] = x1
  ta_ref[...] = jnp.dot(x1, wa_ref[...], preferred_element_type=jnp.float32)
  tb_ref[...] = jnp.dot(x1, wb_ref[...], preferred_element_type=jnp.float32)


def _combine(agg, cnt, r, wa, wb):
  n, d = r.shape
  return pl.pallas_call(
      _combine_body,
      grid=(n // BM,),
      in_specs=[
          pl.BlockSpec((2, BM, d), lambda i: (0, i, 0)),
          pl.BlockSpec((2, BM, d), lambda i: (0, i, 0)),
          pl.BlockSpec((BM, d), lambda i: (i, 0)),
          pl.BlockSpec((d, d), lambda i: (0, 0)),
          pl.BlockSpec((d, d), lambda i: (0, 0)),
      ],
      out_specs=[
          pl.BlockSpec((BM, d), lambda i: (i, 0)),
          pl.BlockSpec((BM, d), lambda i: (i, 0)),
          pl.BlockSpec((BM, d), lambda i: (i, 0)),
      ],
      out_shape=[
          jax.ShapeDtypeStruct((n, d), jnp.float32),
          jax.ShapeDtypeStruct((n, d), jnp.float32),
          jax.ShapeDtypeStruct((n, d), jnp.float32),
      ],
  )(agg, cnt, r, wa, wb)


def _final_body(agg_ref, cnt_ref, r_ref, x1_ref, b_ref, o_ref):
  @pl.when(pl.program_id(0) == 0)
  def _():
    o_ref[...] = jnp.full(o_ref.shape, NEG_INF, jnp.float32)

  ssum = agg_ref[0] + agg_ref[1]
  cnt = jnp.maximum(cnt_ref[0, :, 0:1] + cnt_ref[1, :, 0:1], 1.0)
  x2 = jnp.maximum(ssum / cnt + r_ref[...], 0.0)
  xs = jnp.concatenate([x1_ref[...], x2], axis=1)
  b = b_ref[...]
  gmin = b_ref[0, 0]
  gmax = b_ref[b_ref.shape[0] - 1, 0]
  for g in range(G):
    @pl.when(jnp.logical_and(gmin <= g, g <= gmax))
    def _(g=g):
      col = jnp.max(jnp.where(b == g, xs, NEG_INF), axis=0)
      o_ref[g, :] = jnp.maximum(o_ref[g, :], col)


def _final(agg, cnt, r, x1, batch2):
  n, d = r.shape
  return pl.pallas_call(
      _final_body,
      grid=(n // BM,),
      in_specs=[
          pl.BlockSpec((2, BM, d), lambda i: (0, i, 0)),
          pl.BlockSpec((2, BM, d), lambda i: (0, i, 0)),
          pl.BlockSpec((BM, d), lambda i: (i, 0)),
          pl.BlockSpec((BM, d), lambda i: (i, 0)),
          pl.BlockSpec((BM, 1), lambda i: (i, 0)),
      ],
      out_specs=pl.BlockSpec((G, 2 * d), lambda i: (0, 0)),
      out_shape=jax.ShapeDtypeStruct((G, 2 * d), jnp.float32),
  )(agg, cnt, r, x1, batch2)


def kernel(x, edge_index, batch, W1_l, W1_r, W2_l, W2_r):
  n, d = x.shape
  e = edge_index.shape[1]
  src = edge_index[0]
  dst = edge_index[1]

  agg, np_pad = _sc_aggregate_fn(n, d, e)

  def as3d(a):
    return a.reshape(NC, np_pad, d)

  t1, r1 = _mm2(x, W1_l.T, W1_r.T)
  agg1 = as3d(agg(t1, src, dst))
  cnt = as3d(_sc_count_fn(n, e)(src, dst))
  x1, t2, r2 = _combine(agg1, cnt, r1, W2_l.T, W2_r.T)
  agg2 = as3d(agg(t2, src, dst))
  return _final(agg2, cnt, r2, x1, batch.reshape(n, 1))
